# baseline (speedup 1.0000x reference)
import functools

import jax
import jax.numpy as jnp
from jax import lax
from jax.experimental import pallas as pl
from jax.experimental.pallas import tpu as pltpu
from jax.experimental.pallas import tpu_sc as plsc

NC, NS = 2, 16
NW = NC * NS
CHUNK = 128
NBUF = 4


def _gather_rows(table, idx_flat, B, D):
    b_per_w = B // NW
    n_chunks = b_per_w // CHUNK
    n_groups = n_chunks // NBUF
    assert n_chunks % NBUF == 0 and n_groups >= 3
    mesh = plsc.VectorSubcoreMesh(
        core_axis_name="c", subcore_axis_name="s",
        num_cores=NC, num_subcores=NS)

    @functools.partial(
        pl.kernel,
        out_type=jax.ShapeDtypeStruct((B * D,), jnp.float32),
        mesh=mesh,
        scratch_types=[
            pltpu.VMEM((b_per_w,), jnp.int32),
            pltpu.VMEM((NBUF, CHUNK, 128), jnp.float32),
            pltpu.VMEM((NBUF, CHUNK * D), jnp.float32),
            [pltpu.SemaphoreType.DMA] * NBUF,
            [pltpu.SemaphoreType.DMA] * NBUF,
        ],
    )
    def k(table_hbm, idx_hbm, out_hbm, idx_v, rows_v, stage_v, gsems, osems):
        wid = lax.axis_index("s") * NC + lax.axis_index("c")
        base = wid * b_per_w

        pltpu.sync_copy(idx_hbm.at[pl.ds(base, b_per_w)], idx_v)

        def start_gather(c, s):
            pltpu.async_copy(table_hbm.at[idx_v.at[pl.ds(c * CHUNK, CHUNK)]],
                             rows_v.at[s], gsems[s])

        def wait_gather(c, s):
            pltpu.make_async_copy(
                table_hbm.at[idx_v.at[pl.ds(c * CHUNK, CHUNK)]],
                rows_v.at[s], gsems[s]).wait()

        def compact(s):
            def crow(r, carry):
                for d0 in range(0, D, 16):
                    stage_v[s, pl.ds(r * D + d0, 16)] = \
                        rows_v[s, r, pl.ds(d0, 16)]
                return carry
            lax.fori_loop(0, CHUNK, crow, 0, unroll=8)

        def start_store(c, s):
            pltpu.async_copy(
                stage_v.at[s],
                out_hbm.at[pl.ds((base + c * CHUNK) * D, CHUNK * D)],
                osems[s])

        def wait_store(c, s):
            pltpu.make_async_copy(
                stage_v.at[s],
                out_hbm.at[pl.ds((base + c * CHUNK) * D, CHUNK * D)],
                osems[s]).wait()

        for s in range(NBUF):
            start_gather(s, s)
        for s in range(NBUF):
            wait_gather(s, s)
            compact(s)
            start_store(s, s)
            start_gather(s + NBUF, s)

        def body(j, carry):
            for s in range(NBUF):
                c = j * NBUF + s
                wait_gather(c, s)
                wait_store(c - NBUF, s)
                compact(s)
                start_store(c, s)
                start_gather(c + NBUF, s)
            return carry

        lax.fori_loop(1, n_groups - 1, body, 0, unroll=False)

        for s in range(NBUF):
            c = (n_groups - 1) * NBUF + s
            wait_gather(c, s)
            wait_store(c - NBUF, s)
            compact(s)
            start_store(c, s)
        for s in range(NBUF):
            wait_store((n_groups - 1) * NBUF + s, s)

    return k(table, idx_flat)


def _pad_table_tc(encoding):
    V, D = encoding.shape
    BLK = 1000

    def body(x_ref, o_ref):
        o_ref[:, :D] = x_ref[...]
        o_ref[:, D:] = jnp.zeros((BLK, 128 - D), jnp.float32)

    return pl.pallas_call(
        body,
        grid=(V // BLK,),
        in_specs=[pl.BlockSpec((BLK, D), lambda i: (i, 0))],
        out_specs=pl.BlockSpec((BLK, 128), lambda i: (i, 0)),
        out_shape=jax.ShapeDtypeStruct((V, 128), jnp.float32),
    )(encoding)


def kernel(encoding, ix):
    B = ix.shape[0] * ix.shape[1]
    V, D = encoding.shape
    idx_flat = ix.astype(jnp.int32).reshape(B)
    table_padded = _pad_table_tc(encoding)
    out = _gather_rows(table_padded, idx_flat, B, D)
    return out.reshape(ix.shape[0], ix.shape[1], D)


# speedup vs baseline: 1.4716x; 1.4716x over previous
import functools

import jax
import jax.numpy as jnp
from jax import lax
from jax.experimental import pallas as pl
from jax.experimental.pallas import tpu as pltpu
from jax.experimental.pallas import tpu_sc as plsc

NC, NS = 2, 16
NW = NC * NS
SUBS = (120, 80)
NBUF = 2


def _gather_rows(table, idx_flat, N0, N1, D):
    B = N0 * N1
    b_per_w = B // NW
    rows_w = N0 // NW
    mesh = plsc.VectorSubcoreMesh(
        core_axis_name="c", subcore_axis_name="s",
        num_cores=NC, num_subcores=NS)

    @functools.partial(
        pl.kernel,
        out_type=jax.ShapeDtypeStruct((N0, N1, D), jnp.float32),
        mesh=mesh,
        compiler_params=pltpu.CompilerParams(use_tc_tiling_on_sc=False),
        scratch_types=[
            pltpu.VMEM((b_per_w,), jnp.int32),
            pltpu.VMEM((NBUF, SUBS[0], D), jnp.float32),
            pltpu.VMEM((NBUF, SUBS[1], D), jnp.float32),
            [pltpu.SemaphoreType.DMA] * (2 * NBUF),
            [pltpu.SemaphoreType.DMA] * (2 * NBUF),
        ],
    )
    def k(table_hbm, idx_hbm, out_hbm, idx_v, rows_a, rows_b, gsems, osems):
        wid = lax.axis_index("s") * NC + lax.axis_index("c")
        base_b = wid * rows_w

        pltpu.sync_copy(idx_hbm.at[pl.ds(wid * b_per_w, b_per_w)], idx_v)

        def bufs(h):
            return rows_a if h == 0 else rows_b

        def off(r, h):
            return r * N1 + (SUBS[0] if h else 0)

        def start_gather(r, h, s):
            pltpu.async_copy(
                table_hbm.at[idx_v.at[pl.ds(off(r, h), SUBS[h])]],
                bufs(h).at[s], gsems[2 * s + h])

        def wait_gather(r, h, s):
            pltpu.make_async_copy(
                table_hbm.at[idx_v.at[pl.ds(off(r, h), SUBS[h])]],
                bufs(h).at[s], gsems[2 * s + h]).wait()

        def start_store(r, h, s):
            pltpu.async_copy(
                bufs(h).at[s],
                out_hbm.at[base_b + r].at[
                    pl.ds(SUBS[0] if h else 0, SUBS[h])],
                osems[2 * s + h])

        def wait_store(r, h, s):
            pltpu.make_async_copy(
                bufs(h).at[s],
                out_hbm.at[base_b + r].at[
                    pl.ds(SUBS[0] if h else 0, SUBS[h])],
                osems[2 * s + h]).wait()

        for s in range(NBUF):
            for h in range(2):
                start_gather(s, h, s)
        for h in range(2):
            wait_gather(0, h, 0)
            start_store(0, h, 0)
        for s in range(1, NBUF):
            for h in range(2):
                wait_store(s - 1, h, s - 1)
                start_gather(s - 1 + NBUF, h, s - 1)
                wait_gather(s, h, s)
                start_store(s, h, s)

        def body(j, carry):
            for s in range(NBUF):
                r = j * NBUF + s
                sp = (s - 1) % NBUF
                for h in range(2):
                    wait_store(r - 1, h, sp)
                    start_gather(r - 1 + NBUF, h, sp)
                    wait_gather(r, h, s)
                    start_store(r, h, s)
            return carry

        lax.fori_loop(1, rows_w // NBUF - 1, body, 0, unroll=False)

        for s in range(NBUF):
            r = rows_w - NBUF + s
            sp = (s - 1) % NBUF
            for h in range(2):
                wait_store(r - 1, h, sp)
                if s == 0:
                    start_gather(r - 1 + NBUF, h, sp)
                wait_gather(r, h, s)
                start_store(r, h, s)
        for h in range(2):
            wait_store(rows_w - 1, h, NBUF - 1)

    return k(table, idx_flat)


def kernel(encoding, ix):
    N0, N1 = ix.shape
    V, D = encoding.shape
    idx_flat = ix.astype(jnp.int32).reshape(N0 * N1)
    return _gather_rows(encoding, idx_flat, N0, N1, D)


# speedup vs baseline: 1.4772x; 1.0038x over previous
import functools

import jax
import jax.numpy as jnp
from jax import lax
from jax.experimental import pallas as pl
from jax.experimental.pallas import tpu as pltpu
from jax.experimental.pallas import tpu_sc as plsc

NC, NS = 2, 16
NW = NC * NS
CHUNK = 128
NBUF = 4


def _gather_rows(table, idx_flat, B, D):
    b_per_w = B // NW
    n_chunks = b_per_w // CHUNK
    n_groups = n_chunks // NBUF
    assert n_chunks % NBUF == 0 and n_groups >= 3
    mesh = plsc.VectorSubcoreMesh(
        core_axis_name="c", subcore_axis_name="s",
        num_cores=NC, num_subcores=NS)

    @functools.partial(
        pl.kernel,
        out_type=jax.ShapeDtypeStruct((B, D), jnp.float32),
        mesh=mesh,
        compiler_params=pltpu.CompilerParams(use_tc_tiling_on_sc=False),
        scratch_types=[
            pltpu.VMEM((b_per_w,), jnp.int32),
            pltpu.VMEM((NBUF, CHUNK, D), jnp.float32),
            [pltpu.SemaphoreType.DMA] * NBUF,
            [pltpu.SemaphoreType.DMA] * NBUF,
        ],
    )
    def k(table_hbm, idx_hbm, out_hbm, idx_v, rows_v, gsems, osems):
        wid = lax.axis_index("s") * NC + lax.axis_index("c")
        base = wid * b_per_w

        pltpu.sync_copy(idx_hbm.at[pl.ds(base, b_per_w)], idx_v)

        def start_gather(c, s):
            pltpu.async_copy(table_hbm.at[idx_v.at[pl.ds(c * CHUNK, CHUNK)]],
                             rows_v.at[s], gsems[s])

        def wait_gather(c, s):
            pltpu.make_async_copy(
                table_hbm.at[idx_v.at[pl.ds(c * CHUNK, CHUNK)]],
                rows_v.at[s], gsems[s]).wait()

        def start_store(c, s):
            pltpu.async_copy(rows_v.at[s],
                             out_hbm.at[pl.ds(base + c * CHUNK, CHUNK)],
                             osems[s])

        def wait_store(c, s):
            pltpu.make_async_copy(
                rows_v.at[s],
                out_hbm.at[pl.ds(base + c * CHUNK, CHUNK)],
                osems[s]).wait()

        for s in range(NBUF):
            start_gather(s, s)
        wait_gather(0, 0)
        start_store(0, 0)
        for s in range(1, NBUF):
            wait_store(s - 1, s - 1)
            start_gather(s - 1 + NBUF, s - 1)
            wait_gather(s, s)
            start_store(s, s)

        def body(j, carry):
            for s in range(NBUF):
                c = j * NBUF + s
                sp = (s - 1) % NBUF
                wait_store(c - 1, sp)
                start_gather(c - 1 + NBUF, sp)
                wait_gather(c, s)
                start_store(c, s)
            return carry

        lax.fori_loop(1, n_groups - 1, body, 0, unroll=False)

        for s in range(NBUF):
            c = (n_groups - 1) * NBUF + s
            sp = (s - 1) % NBUF
            wait_store(c - 1, sp)
            if s == 0:
                start_gather(c - 1 + NBUF, sp)
            wait_gather(c, s)
            start_store(c, s)
        wait_store(n_chunks - 1, NBUF - 1)

    return k(table, idx_flat)


def kernel(encoding, ix):
    B = ix.shape[0] * ix.shape[1]
    D = encoding.shape[1]
    idx_flat = ix.astype(jnp.int32).reshape(B)
    out = _gather_rows(encoding, idx_flat, B, D)
    return out.reshape(ix.shape[0], ix.shape[1], D)
